# baseline (speedup 1.0000x reference)
import functools

import jax
import jax.numpy as jnp
from jax import lax
from jax.experimental import pallas as pl
from jax.experimental.pallas import tpu as pltpu
from jax.experimental.pallas import tpu_sc as plsc

_N = 10000
_E = 320000
_HID = 256
_F = 16
_NS = 16
_NC = 2
_NH = 5120
_SKIP = 640
_SKW = 16
_AR = _NH + _SKW
_B = 128
_BD = 80
_R = 1000
_DW = 16


def _agg_body(gt_ref, srcg_ref, dst2_ref, out_ref, srcb, dstb, rows, sem,
              *accs):
    c = lax.axis_index("c")
    s = lax.axis_index("s")
    zero16 = jnp.zeros((16,), jnp.float32)

    def zrow(i, _):
        rows[i, :] = zero16
        return 0

    lax.fori_loop(0, _B, zrow, 0)
    for t in range(_NS):
        @pl.when(s == t)
        def _():
            for k in range(_AR // _B):
                pltpu.sync_copy(rows, accs[t].at[pl.ds(k * _B, _B)])
            pltpu.sync_copy(rows.at[pl.ds(0, _AR % _B)],
                            accs[t].at[pl.ds(_AR - _AR % _B, _AR % _B)])

    def blk(b, _):
        e0 = b * _B
        pltpu.sync_copy(srcg_ref.at[s, pl.ds(e0, _B)], srcb)
        pltpu.sync_copy(dst2_ref.at[c, pl.ds(e0, _B)], dstb)
        pltpu.async_copy(gt_ref.at[srcb], rows, sem).wait()
        for t in range(_NS):
            @pl.when(s == t)
            def _():
                pltpu.sync_copy(rows, accs[t].at[dstb], add=True)
        return 0

    lax.fori_loop(0, _E // _B, blk, 0)

    for t in range(_NS):
        @pl.when(s == t)
        def _():
            pltpu.sync_copy(accs[t].at[pl.ds(0, _SKIP)],
                            out_ref.at[c, t, pl.ds(0, _SKIP)])
            pltpu.sync_copy(accs[t].at[pl.ds(_SKIP + _SKW, _NH - _SKIP)],
                            out_ref.at[c, t, pl.ds(_SKIP, _NH - _SKIP)])


@functools.cache
def _get_agg():
    return pl.kernel(
        _agg_body,
        out_type=jax.ShapeDtypeStruct((_NC, _NS, _NH, _F), jnp.float32),
        mesh=plsc.VectorSubcoreMesh(core_axis_name="c", subcore_axis_name="s",
                                    num_cores=_NC, num_subcores=_NS),
        scratch_types=[
            pltpu.VMEM((_B,), jnp.int32),
            pltpu.VMEM((_B,), jnp.int32),
            pltpu.VMEM((_B, _F), jnp.float32),
            pltpu.SemaphoreType.DMA,
        ] + [pltpu.VMEM_SHARED((_AR, _F), jnp.float32)] * _NS,
        compiler_params=pltpu.CompilerParams(use_tc_tiling_on_sc=False),
    )


def _deg_body(dst2_ref, out_ref, dstb, ones, acc):
    c = lax.axis_index("c")
    s = lax.axis_index("s")
    one16 = jnp.ones((16,), jnp.float32)

    def orow(i, _):
        ones[i, :] = one16
        return 0

    lax.fori_loop(0, _B, orow, 0)

    @pl.when(s == 0)
    def _():
        def zrow(i, _):
            ones[i, :] = jnp.zeros((16,), jnp.float32)
            return 0

        lax.fori_loop(0, _B, zrow, 0)
        for k in range(_AR // _B):
            pltpu.sync_copy(ones, acc.at[pl.ds(k * _B, _B)])
        pltpu.sync_copy(ones.at[pl.ds(0, _AR % _B)],
                        acc.at[pl.ds(_AR - _AR % _B, _AR % _B)])

        def orow2(i, _):
            ones[i, :] = one16
            return 0

        lax.fori_loop(0, _B, orow2, 0)

    plsc.subcore_barrier()
    ept = _E // _NS

    def blk(b, _):
        e0 = s * ept + b * _BD
        pltpu.sync_copy(dst2_ref.at[c, pl.ds(e0, _BD)], dstb)
        pltpu.sync_copy(ones.at[pl.ds(0, _BD)], acc.at[dstb], add=True)
        return 0

    lax.fori_loop(0, ept // _BD, blk, 0)
    plsc.subcore_barrier()

    @pl.when(s == 0)
    def _():
        pltpu.sync_copy(acc.at[pl.ds(0, _SKIP)], out_ref.at[c, pl.ds(0, _SKIP)])

    @pl.when(s == 1)
    def _():
        pltpu.sync_copy(acc.at[pl.ds(_SKIP + _SKW, _NH - _SKIP)],
                        out_ref.at[c, pl.ds(_SKIP, _NH - _SKIP)])


@functools.cache
def _get_deg():
    return pl.kernel(
        _deg_body,
        out_type=jax.ShapeDtypeStruct((_NC, _NH, _DW), jnp.float32),
        mesh=plsc.VectorSubcoreMesh(core_axis_name="c", subcore_axis_name="s",
                                    num_cores=_NC, num_subcores=_NS),
        scratch_types=[
            pltpu.VMEM((_BD,), jnp.int32),
            pltpu.VMEM((_B, _DW), jnp.float32),
            pltpu.VMEM_SHARED((_AR, _DW), jnp.float32),
        ],
        compiler_params=pltpu.CompilerParams(use_tc_tiling_on_sc=False),
    )


def _split_g(g, g_ref):
    for fg in range(_NS):
        g_ref[fg] = g[:, fg * _F:(fg + 1) * _F]


def _enc_body(x_ref, W1_ref, b1_ref, W2_ref, b2_ref, Wn_ref, h_ref, g_ref):
    h = jnp.dot(x_ref[...], W1_ref[...], preferred_element_type=jnp.float32)
    h = jnp.maximum(h + b1_ref[...], 0.0)
    h = jnp.dot(h, W2_ref[...], preferred_element_type=jnp.float32)
    h = jnp.maximum(h + b2_ref[...], 0.0)
    h_ref[...] = h
    _split_g(jnp.dot(h, Wn_ref[...], preferred_element_type=jnp.float32),
             g_ref)


def _enc(x, W1, b1, W2, b2, Wn):
    return pl.pallas_call(
        _enc_body,
        grid=(_N // _R,),
        in_specs=[
            pl.BlockSpec((_R, 128), lambda i: (i, 0)),
            pl.BlockSpec((128, _HID), lambda i: (0, 0)),
            pl.BlockSpec((1, _HID), lambda i: (0, 0)),
            pl.BlockSpec((_HID, _HID), lambda i: (0, 0)),
            pl.BlockSpec((1, _HID), lambda i: (0, 0)),
            pl.BlockSpec((_HID, _HID), lambda i: (0, 0)),
        ],
        out_specs=[
            pl.BlockSpec((_R, _HID), lambda i: (i, 0)),
            pl.BlockSpec((_NS, _R, _F), lambda i: (0, i, 0)),
        ],
        out_shape=[
            jax.ShapeDtypeStruct((_N, _HID), jnp.float32),
            jax.ShapeDtypeStruct((_NS, _N, _F), jnp.float32),
        ],
    )(x, W1, b1.reshape(1, -1), W2, b2.reshape(1, -1), Wn)


def _mid_body(h_ref, s_ref, d_ref, Ws_ref, bg_ref, Wn_ref, h1_ref, g_ref):
    rdeg = 1.0 / jnp.maximum(d_ref[...][:, :1], 1.0)
    agg = s_ref[...] * rdeg
    h1 = jnp.dot(h_ref[...], Ws_ref[...], preferred_element_type=jnp.float32)
    h1 = jnp.maximum(h1 + agg + bg_ref[...], 0.0)
    h1_ref[...] = h1
    _split_g(jnp.dot(h1, Wn_ref[...], preferred_element_type=jnp.float32),
             g_ref)


def _mid(h, s, d, Ws, bg, Wn):
    return pl.pallas_call(
        _mid_body,
        grid=(_N // _R,),
        in_specs=[
            pl.BlockSpec((_R, _HID), lambda i: (i, 0)),
            pl.BlockSpec((_R, _HID), lambda i: (i, 0)),
            pl.BlockSpec((_R, _DW), lambda i: (i, 0)),
            pl.BlockSpec((_HID, _HID), lambda i: (0, 0)),
            pl.BlockSpec((1, _HID), lambda i: (0, 0)),
            pl.BlockSpec((_HID, _HID), lambda i: (0, 0)),
        ],
        out_specs=[
            pl.BlockSpec((_R, _HID), lambda i: (i, 0)),
            pl.BlockSpec((_NS, _R, _F), lambda i: (0, i, 0)),
        ],
        out_shape=[
            jax.ShapeDtypeStruct((_N, _HID), jnp.float32),
            jax.ShapeDtypeStruct((_NS, _N, _F), jnp.float32),
        ],
    )(h, s, d, Ws, bg.reshape(1, -1), Wn)


def _fin_body(h_ref, s_ref, d_ref, Ws_ref, bg_ref, Wc1_ref, bc1_ref, Wc2_ref,
              bc2_ref, out_ref, emb_ref):
    rdeg = 1.0 / jnp.maximum(d_ref[...][:, :1], 1.0)
    agg = s_ref[...] * rdeg
    emb = jnp.dot(h_ref[...], Ws_ref[...], preferred_element_type=jnp.float32)
    emb = jnp.maximum(emb + agg + bg_ref[...], 0.0)
    emb_ref[...] = emb
    t = jnp.dot(emb, Wc1_ref[...], preferred_element_type=jnp.float32)
    t = jnp.maximum(t + bc1_ref[...], 0.0)
    out_ref[...] = (
        jnp.dot(t, Wc2_ref[...], preferred_element_type=jnp.float32)
        + bc2_ref[...])


def _fin(h, s, d, Ws, bg, Wc1, bc1, Wc2, bc2):
    return pl.pallas_call(
        _fin_body,
        grid=(_N // _R,),
        in_specs=[
            pl.BlockSpec((_R, _HID), lambda i: (i, 0)),
            pl.BlockSpec((_R, _HID), lambda i: (i, 0)),
            pl.BlockSpec((_R, _DW), lambda i: (i, 0)),
            pl.BlockSpec((_HID, _HID), lambda i: (0, 0)),
            pl.BlockSpec((1, _HID), lambda i: (0, 0)),
            pl.BlockSpec((_HID, _HID), lambda i: (0, 0)),
            pl.BlockSpec((1, _HID), lambda i: (0, 0)),
            pl.BlockSpec((_HID, 128), lambda i: (0, 0)),
            pl.BlockSpec((1, 128), lambda i: (0, 0)),
        ],
        out_specs=[
            pl.BlockSpec((_R, 128), lambda i: (i, 0)),
            pl.BlockSpec((_R, _HID), lambda i: (i, 0)),
        ],
        out_shape=[
            jax.ShapeDtypeStruct((_N, 128), jnp.float32),
            jax.ShapeDtypeStruct((_N, _HID), jnp.float32),
        ],
    )(h, s, d, Ws, bg.reshape(1, -1), Wc1, bc1.reshape(1, -1), Wc2,
      bc2.reshape(1, -1))


def _glue_s(sagg):
    return sagg.transpose(0, 2, 1, 3).reshape(_NC * _NH, _HID)


def kernel(x, edge_index, W1, b1, W2, b2, Ws0, Wn0, bg0, Ws1, Wn1, bg1, Ws2,
           Wn2, bg2, Wc1, bc1, Wc2, bc2):
    _agg = _get_agg()
    _deg = _get_deg()
    src = edge_index[0]
    dst = edge_index[1]
    srcg = jnp.arange(_NS, dtype=jnp.int32)[:, None] * _N + src[None, :]
    dummy = _SKIP + (jnp.arange(_E, dtype=jnp.int32) % 8)
    d2 = []
    for c in range(_NC):
        local = dst - c * _NH
        inh = (local >= 0) & (local < _NH)
        mapped = local + jnp.where(local >= _SKIP, _SKW, 0)
        d2.append(jnp.where(inh, mapped, dummy))
    dst2 = jnp.stack(d2)

    d = _deg(dst2).reshape(_NC * _NH, _DW)
    h, g = _enc(x, W1, b1, W2, b2, Wn0)
    s = _glue_s(_agg(g.reshape(_NS * _N, _F), srcg, dst2))
    h, g = _mid(h, s, d, Ws0, bg0, Wn1)
    s = _glue_s(_agg(g.reshape(_NS * _N, _F), srcg, dst2))
    h, g = _mid(h, s, d, Ws1, bg1, Wn2)
    s = _glue_s(_agg(g.reshape(_NS * _N, _F), srcg, dst2))
    out, emb = _fin(h, s, d, Ws2, bg2, Wc1, bc1, Wc2, bc2)
    return (out, emb)


# speedup vs baseline: 1.5704x; 1.5704x over previous
import functools

import jax
import jax.numpy as jnp
from jax import lax
from jax.experimental import pallas as pl
from jax.experimental.pallas import tpu as pltpu
from jax.experimental.pallas import tpu_sc as plsc

_N = 10000
_E = 320000
_HID = 256
_F = 16
_NS = 16
_NC = 2
_NH = 5120
_SKIP = 640
_SKW = 16
_AR = _NH + _SKW
_B = 128
_BD = 80
_R = 1000
_DW = 16


def _agg_body(gt_ref, srcg_ref, dst2_ref, out_ref, srcb0, dstb0, rows0, srcb1,
              dstb1, rows1, sem, semg0, semg1, *accs):
    c = lax.axis_index("c")
    s = lax.axis_index("s")
    zero16 = jnp.zeros((16,), jnp.float32)

    def zrow(i, _):
        rows0[i, :] = zero16
        return 0

    lax.fori_loop(0, _B, zrow, 0)
    for t in range(_NS):
        @pl.when(s == t)
        def _():
            for k in range(_AR // _B):
                pltpu.sync_copy(rows0, accs[t].at[pl.ds(k * _B, _B)])
            pltpu.sync_copy(rows0.at[pl.ds(0, _AR % _B)],
                            accs[t].at[pl.ds(_AR - _AR % _B, _AR % _B)])

    bufs = ((srcb0, dstb0, rows0, semg0), (srcb1, dstb1, rows1, semg1))
    nblk = _E // _B

    pltpu.sync_copy(srcg_ref.at[s, pl.ds(0, _B)], srcb0)
    pltpu.sync_copy(dst2_ref.at[c, pl.ds(0, _B)], dstb0)
    pltpu.async_copy(gt_ref.at[srcb0], rows0, semg0)

    def pair(k, _):
        for p in range(2):
            sb, db, rw, sg = bufs[p]
            nb, ndb, nrw, nsg = bufs[1 - p]
            b = 2 * k + p
            @pl.when(b + 1 < nblk)
            def _():
                e1 = (b + 1) * _B
                pltpu.sync_copy(srcg_ref.at[s, pl.ds(e1, _B)], nb)
                pltpu.sync_copy(dst2_ref.at[c, pl.ds(e1, _B)], ndb)
                pltpu.async_copy(gt_ref.at[nb], nrw, nsg)

            pltpu.make_async_copy(gt_ref.at[sb], rw, sg).wait()
            for t in range(_NS):
                @pl.when(s == t)
                def _():
                    pltpu.sync_copy(rw, accs[t].at[db], add=True)
        return 0

    lax.fori_loop(0, nblk // 2, pair, 0)

    for t in range(_NS):
        @pl.when(s == t)
        def _():
            pltpu.sync_copy(accs[t].at[pl.ds(0, _SKIP)],
                            out_ref.at[c, t, pl.ds(0, _SKIP)])
            pltpu.sync_copy(accs[t].at[pl.ds(_SKIP + _SKW, _NH - _SKIP)],
                            out_ref.at[c, t, pl.ds(_SKIP, _NH - _SKIP)])


@functools.cache
def _get_agg():
    return pl.kernel(
        _agg_body,
        out_type=jax.ShapeDtypeStruct((_NC, _NS, _NH, _F), jnp.float32),
        mesh=plsc.VectorSubcoreMesh(core_axis_name="c", subcore_axis_name="s",
                                    num_cores=_NC, num_subcores=_NS),
        scratch_types=[
            pltpu.VMEM((_B,), jnp.int32),
            pltpu.VMEM((_B,), jnp.int32),
            pltpu.VMEM((_B, _F), jnp.float32),
            pltpu.VMEM((_B,), jnp.int32),
            pltpu.VMEM((_B,), jnp.int32),
            pltpu.VMEM((_B, _F), jnp.float32),
            pltpu.SemaphoreType.DMA,
            pltpu.SemaphoreType.DMA,
            pltpu.SemaphoreType.DMA,
        ] + [pltpu.VMEM_SHARED((_AR, _F), jnp.float32)] * _NS,
        compiler_params=pltpu.CompilerParams(use_tc_tiling_on_sc=False),
    )


def _deg_body(dst2_ref, out_ref, dstb, ones, acc):
    c = lax.axis_index("c")
    s = lax.axis_index("s")
    one16 = jnp.ones((16,), jnp.float32)

    def orow(i, _):
        ones[i, :] = one16
        return 0

    lax.fori_loop(0, _B, orow, 0)

    @pl.when(s == 0)
    def _():
        def zrow(i, _):
            ones[i, :] = jnp.zeros((16,), jnp.float32)
            return 0

        lax.fori_loop(0, _B, zrow, 0)
        for k in range(_AR // _B):
            pltpu.sync_copy(ones, acc.at[pl.ds(k * _B, _B)])
        pltpu.sync_copy(ones.at[pl.ds(0, _AR % _B)],
                        acc.at[pl.ds(_AR - _AR % _B, _AR % _B)])

        def orow2(i, _):
            ones[i, :] = one16
            return 0

        lax.fori_loop(0, _B, orow2, 0)

    plsc.subcore_barrier()
    ept = _E // _NS

    def blk(b, _):
        e0 = s * ept + b * _BD
        pltpu.sync_copy(dst2_ref.at[c, pl.ds(e0, _BD)], dstb)
        pltpu.sync_copy(ones.at[pl.ds(0, _BD)], acc.at[dstb], add=True)
        return 0

    lax.fori_loop(0, ept // _BD, blk, 0)
    plsc.subcore_barrier()

    @pl.when(s == 0)
    def _():
        pltpu.sync_copy(acc.at[pl.ds(0, _SKIP)], out_ref.at[c, pl.ds(0, _SKIP)])

    @pl.when(s == 1)
    def _():
        pltpu.sync_copy(acc.at[pl.ds(_SKIP + _SKW, _NH - _SKIP)],
                        out_ref.at[c, pl.ds(_SKIP, _NH - _SKIP)])


@functools.cache
def _get_deg():
    return pl.kernel(
        _deg_body,
        out_type=jax.ShapeDtypeStruct((_NC, _NH, _DW), jnp.float32),
        mesh=plsc.VectorSubcoreMesh(core_axis_name="c", subcore_axis_name="s",
                                    num_cores=_NC, num_subcores=_NS),
        scratch_types=[
            pltpu.VMEM((_BD,), jnp.int32),
            pltpu.VMEM((_B, _DW), jnp.float32),
            pltpu.VMEM_SHARED((_AR, _DW), jnp.float32),
        ],
        compiler_params=pltpu.CompilerParams(use_tc_tiling_on_sc=False),
    )


def _split_g(g, g_ref):
    for fg in range(_NS):
        g_ref[fg] = g[:, fg * _F:(fg + 1) * _F]


def _enc_body(x_ref, W1_ref, b1_ref, W2_ref, b2_ref, Wn_ref, h_ref, g_ref):
    h = jnp.dot(x_ref[...], W1_ref[...], preferred_element_type=jnp.float32)
    h = jnp.maximum(h + b1_ref[...], 0.0)
    h = jnp.dot(h, W2_ref[...], preferred_element_type=jnp.float32)
    h = jnp.maximum(h + b2_ref[...], 0.0)
    h_ref[...] = h
    _split_g(jnp.dot(h, Wn_ref[...], preferred_element_type=jnp.float32),
             g_ref)


def _enc(x, W1, b1, W2, b2, Wn):
    return pl.pallas_call(
        _enc_body,
        grid=(_N // _R,),
        in_specs=[
            pl.BlockSpec((_R, 128), lambda i: (i, 0)),
            pl.BlockSpec((128, _HID), lambda i: (0, 0)),
            pl.BlockSpec((1, _HID), lambda i: (0, 0)),
            pl.BlockSpec((_HID, _HID), lambda i: (0, 0)),
            pl.BlockSpec((1, _HID), lambda i: (0, 0)),
            pl.BlockSpec((_HID, _HID), lambda i: (0, 0)),
        ],
        out_specs=[
            pl.BlockSpec((_R, _HID), lambda i: (i, 0)),
            pl.BlockSpec((_NS, _R, _F), lambda i: (0, i, 0)),
        ],
        out_shape=[
            jax.ShapeDtypeStruct((_N, _HID), jnp.float32),
            jax.ShapeDtypeStruct((_NS, _N, _F), jnp.float32),
        ],
    )(x, W1, b1.reshape(1, -1), W2, b2.reshape(1, -1), Wn)


def _mid_body(h_ref, s_ref, d_ref, Ws_ref, bg_ref, Wn_ref, h1_ref, g_ref):
    rdeg = 1.0 / jnp.maximum(d_ref[...][:, :1], 1.0)
    agg = s_ref[...] * rdeg
    h1 = jnp.dot(h_ref[...], Ws_ref[...], preferred_element_type=jnp.float32)
    h1 = jnp.maximum(h1 + agg + bg_ref[...], 0.0)
    h1_ref[...] = h1
    _split_g(jnp.dot(h1, Wn_ref[...], preferred_element_type=jnp.float32),
             g_ref)


def _mid(h, s, d, Ws, bg, Wn):
    return pl.pallas_call(
        _mid_body,
        grid=(_N // _R,),
        in_specs=[
            pl.BlockSpec((_R, _HID), lambda i: (i, 0)),
            pl.BlockSpec((_R, _HID), lambda i: (i, 0)),
            pl.BlockSpec((_R, _DW), lambda i: (i, 0)),
            pl.BlockSpec((_HID, _HID), lambda i: (0, 0)),
            pl.BlockSpec((1, _HID), lambda i: (0, 0)),
            pl.BlockSpec((_HID, _HID), lambda i: (0, 0)),
        ],
        out_specs=[
            pl.BlockSpec((_R, _HID), lambda i: (i, 0)),
            pl.BlockSpec((_NS, _R, _F), lambda i: (0, i, 0)),
        ],
        out_shape=[
            jax.ShapeDtypeStruct((_N, _HID), jnp.float32),
            jax.ShapeDtypeStruct((_NS, _N, _F), jnp.float32),
        ],
    )(h, s, d, Ws, bg.reshape(1, -1), Wn)


def _fin_body(h_ref, s_ref, d_ref, Ws_ref, bg_ref, Wc1_ref, bc1_ref, Wc2_ref,
              bc2_ref, out_ref, emb_ref):
    rdeg = 1.0 / jnp.maximum(d_ref[...][:, :1], 1.0)
    agg = s_ref[...] * rdeg
    emb = jnp.dot(h_ref[...], Ws_ref[...], preferred_element_type=jnp.float32)
    emb = jnp.maximum(emb + agg + bg_ref[...], 0.0)
    emb_ref[...] = emb
    t = jnp.dot(emb, Wc1_ref[...], preferred_element_type=jnp.float32)
    t = jnp.maximum(t + bc1_ref[...], 0.0)
    out_ref[...] = (
        jnp.dot(t, Wc2_ref[...], preferred_element_type=jnp.float32)
        + bc2_ref[...])


def _fin(h, s, d, Ws, bg, Wc1, bc1, Wc2, bc2):
    return pl.pallas_call(
        _fin_body,
        grid=(_N // _R,),
        in_specs=[
            pl.BlockSpec((_R, _HID), lambda i: (i, 0)),
            pl.BlockSpec((_R, _HID), lambda i: (i, 0)),
            pl.BlockSpec((_R, _DW), lambda i: (i, 0)),
            pl.BlockSpec((_HID, _HID), lambda i: (0, 0)),
            pl.BlockSpec((1, _HID), lambda i: (0, 0)),
            pl.BlockSpec((_HID, _HID), lambda i: (0, 0)),
            pl.BlockSpec((1, _HID), lambda i: (0, 0)),
            pl.BlockSpec((_HID, 128), lambda i: (0, 0)),
            pl.BlockSpec((1, 128), lambda i: (0, 0)),
        ],
        out_specs=[
            pl.BlockSpec((_R, 128), lambda i: (i, 0)),
            pl.BlockSpec((_R, _HID), lambda i: (i, 0)),
        ],
        out_shape=[
            jax.ShapeDtypeStruct((_N, 128), jnp.float32),
            jax.ShapeDtypeStruct((_N, _HID), jnp.float32),
        ],
    )(h, s, d, Ws, bg.reshape(1, -1), Wc1, bc1.reshape(1, -1), Wc2,
      bc2.reshape(1, -1))


def _glue_s(sagg):
    return sagg.transpose(0, 2, 1, 3).reshape(_NC * _NH, _HID)


def kernel(x, edge_index, W1, b1, W2, b2, Ws0, Wn0, bg0, Ws1, Wn1, bg1, Ws2,
           Wn2, bg2, Wc1, bc1, Wc2, bc2):
    _agg = _get_agg()
    _deg = _get_deg()
    src = edge_index[0]
    dst = edge_index[1]
    srcg = jnp.arange(_NS, dtype=jnp.int32)[:, None] * _N + src[None, :]
    dummy = _SKIP + (jnp.arange(_E, dtype=jnp.int32) % 8)
    d2 = []
    for c in range(_NC):
        local = dst - c * _NH
        inh = (local >= 0) & (local < _NH)
        mapped = local + jnp.where(local >= _SKIP, _SKW, 0)
        d2.append(jnp.where(inh, mapped, dummy))
    dst2 = jnp.stack(d2)

    d = _deg(dst2).reshape(_NC * _NH, _DW)
    h, g = _enc(x, W1, b1, W2, b2, Wn0)
    s = _glue_s(_agg(g.reshape(_NS * _N, _F), srcg, dst2))
    h, g = _mid(h, s, d, Ws0, bg0, Wn1)
    s = _glue_s(_agg(g.reshape(_NS * _N, _F), srcg, dst2))
    h, g = _mid(h, s, d, Ws1, bg1, Wn2)
    s = _glue_s(_agg(g.reshape(_NS * _N, _F), srcg, dst2))
    out, emb = _fin(h, s, d, Ws2, bg2, Wc1, bc1, Wc2, bc2)
    return (out, emb)


# speedup vs baseline: 1.5966x; 1.0167x over previous
import functools

import jax
import jax.numpy as jnp
from jax import lax
from jax.experimental import pallas as pl
from jax.experimental.pallas import tpu as pltpu
from jax.experimental.pallas import tpu_sc as plsc

_N = 10000
_E = 320000
_HID = 256
_F = 16
_NS = 16
_NC = 2
_NH = 5120
_SKIP = 640
_SKW = 16
_AR = _NH + _SKW
_B = 128
_BD = 80
_R = 1000
_DW = 16


def _agg_body(gt_ref, srcg_ref, dst2_ref, out_ref, srcb0, dstb0, rows0, srcb1,
              dstb1, rows1, semc0, semg0, semg1, semc1, *accs):
    c = lax.axis_index("c")
    s = lax.axis_index("s")
    zero16 = jnp.zeros((16,), jnp.float32)

    def zrow(i, _):
        rows0[i, :] = zero16
        return 0

    lax.fori_loop(0, _B, zrow, 0)
    for t in range(_NS):
        @pl.when(s == t)
        def _():
            for k in range(_AR // _B):
                pltpu.sync_copy(rows0, accs[t].at[pl.ds(k * _B, _B)])
            pltpu.sync_copy(rows0.at[pl.ds(0, _AR % _B)],
                            accs[t].at[pl.ds(_AR - _AR % _B, _AR % _B)])

    bufs = ((srcb0, dstb0, rows0, semg0, semc0),
            (srcb1, dstb1, rows1, semg1, semc1))
    nblk = _E // _B

    pltpu.sync_copy(srcg_ref.at[s, pl.ds(0, _B)], srcb0)
    pltpu.sync_copy(dst2_ref.at[c, pl.ds(0, _B)], dstb0)
    pltpu.async_copy(gt_ref.at[srcb0], rows0, semg0)

    def pair(k, _):
        for p in range(2):
            sb, db, rw, sg, sc = bufs[p]
            nb, ndb, nrw, nsg, nsc = bufs[1 - p]
            b = 2 * k + p

            @pl.when(b >= 1)
            def _():
                for t in range(_NS):
                    @pl.when(s == t)
                    def _():
                        pltpu.make_async_copy(nrw, accs[t].at[ndb], nsc).wait()

            @pl.when(b + 1 < nblk)
            def _():
                e1 = (b + 1) * _B
                pltpu.sync_copy(srcg_ref.at[s, pl.ds(e1, _B)], nb)
                pltpu.sync_copy(dst2_ref.at[c, pl.ds(e1, _B)], ndb)
                pltpu.async_copy(gt_ref.at[nb], nrw, nsg)

            pltpu.make_async_copy(gt_ref.at[sb], rw, sg).wait()
            for t in range(_NS):
                @pl.when(s == t)
                def _():
                    pltpu.async_copy(rw, accs[t].at[db], sc, add=True)
        return 0

    lax.fori_loop(0, nblk // 2, pair, 0)
    sb, db, rw, sg, sc = bufs[1]
    for t in range(_NS):
        @pl.when(s == t)
        def _():
            pltpu.make_async_copy(rw, accs[t].at[db], sc).wait()

    for t in range(_NS):
        @pl.when(s == t)
        def _():
            pltpu.sync_copy(accs[t].at[pl.ds(0, _SKIP)],
                            out_ref.at[c, t, pl.ds(0, _SKIP)])
            pltpu.sync_copy(accs[t].at[pl.ds(_SKIP + _SKW, _NH - _SKIP)],
                            out_ref.at[c, t, pl.ds(_SKIP, _NH - _SKIP)])


@functools.cache
def _get_agg():
    return pl.kernel(
        _agg_body,
        out_type=jax.ShapeDtypeStruct((_NC, _NS, _NH, _F), jnp.float32),
        mesh=plsc.VectorSubcoreMesh(core_axis_name="c", subcore_axis_name="s",
                                    num_cores=_NC, num_subcores=_NS),
        scratch_types=[
            pltpu.VMEM((_B,), jnp.int32),
            pltpu.VMEM((_B,), jnp.int32),
            pltpu.VMEM((_B, _F), jnp.float32),
            pltpu.VMEM((_B,), jnp.int32),
            pltpu.VMEM((_B,), jnp.int32),
            pltpu.VMEM((_B, _F), jnp.float32),
            pltpu.SemaphoreType.DMA,
            pltpu.SemaphoreType.DMA,
            pltpu.SemaphoreType.DMA,
            pltpu.SemaphoreType.DMA,
        ] + [pltpu.VMEM_SHARED((_AR, _F), jnp.float32)] * _NS,
        compiler_params=pltpu.CompilerParams(use_tc_tiling_on_sc=False),
    )


def _deg_body(dst2_ref, out_ref, dstb, ones, acc):
    c = lax.axis_index("c")
    s = lax.axis_index("s")
    one16 = jnp.ones((16,), jnp.float32)

    def orow(i, _):
        ones[i, :] = one16
        return 0

    lax.fori_loop(0, _B, orow, 0)

    @pl.when(s == 0)
    def _():
        def zrow(i, _):
            ones[i, :] = jnp.zeros((16,), jnp.float32)
            return 0

        lax.fori_loop(0, _B, zrow, 0)
        for k in range(_AR // _B):
            pltpu.sync_copy(ones, acc.at[pl.ds(k * _B, _B)])
        pltpu.sync_copy(ones.at[pl.ds(0, _AR % _B)],
                        acc.at[pl.ds(_AR - _AR % _B, _AR % _B)])

        def orow2(i, _):
            ones[i, :] = one16
            return 0

        lax.fori_loop(0, _B, orow2, 0)

    plsc.subcore_barrier()
    ept = _E // _NS

    def blk(b, _):
        e0 = s * ept + b * _BD
        pltpu.sync_copy(dst2_ref.at[c, pl.ds(e0, _BD)], dstb)
        pltpu.sync_copy(ones.at[pl.ds(0, _BD)], acc.at[dstb], add=True)
        return 0

    lax.fori_loop(0, ept // _BD, blk, 0)
    plsc.subcore_barrier()

    @pl.when(s == 0)
    def _():
        pltpu.sync_copy(acc.at[pl.ds(0, _SKIP)], out_ref.at[c, pl.ds(0, _SKIP)])

    @pl.when(s == 1)
    def _():
        pltpu.sync_copy(acc.at[pl.ds(_SKIP + _SKW, _NH - _SKIP)],
                        out_ref.at[c, pl.ds(_SKIP, _NH - _SKIP)])


@functools.cache
def _get_deg():
    return pl.kernel(
        _deg_body,
        out_type=jax.ShapeDtypeStruct((_NC, _NH, _DW), jnp.float32),
        mesh=plsc.VectorSubcoreMesh(core_axis_name="c", subcore_axis_name="s",
                                    num_cores=_NC, num_subcores=_NS),
        scratch_types=[
            pltpu.VMEM((_BD,), jnp.int32),
            pltpu.VMEM((_B, _DW), jnp.float32),
            pltpu.VMEM_SHARED((_AR, _DW), jnp.float32),
        ],
        compiler_params=pltpu.CompilerParams(use_tc_tiling_on_sc=False),
    )


def _split_g(g, g_ref):
    for fg in range(_NS):
        g_ref[fg] = g[:, fg * _F:(fg + 1) * _F]


def _enc_body(x_ref, W1_ref, b1_ref, W2_ref, b2_ref, Wn_ref, h_ref, g_ref):
    h = jnp.dot(x_ref[...], W1_ref[...], preferred_element_type=jnp.float32)
    h = jnp.maximum(h + b1_ref[...], 0.0)
    h = jnp.dot(h, W2_ref[...], preferred_element_type=jnp.float32)
    h = jnp.maximum(h + b2_ref[...], 0.0)
    h_ref[...] = h
    _split_g(jnp.dot(h, Wn_ref[...], preferred_element_type=jnp.float32),
             g_ref)


def _enc(x, W1, b1, W2, b2, Wn):
    return pl.pallas_call(
        _enc_body,
        grid=(_N // _R,),
        in_specs=[
            pl.BlockSpec((_R, 128), lambda i: (i, 0)),
            pl.BlockSpec((128, _HID), lambda i: (0, 0)),
            pl.BlockSpec((1, _HID), lambda i: (0, 0)),
            pl.BlockSpec((_HID, _HID), lambda i: (0, 0)),
            pl.BlockSpec((1, _HID), lambda i: (0, 0)),
            pl.BlockSpec((_HID, _HID), lambda i: (0, 0)),
        ],
        out_specs=[
            pl.BlockSpec((_R, _HID), lambda i: (i, 0)),
            pl.BlockSpec((_NS, _R, _F), lambda i: (0, i, 0)),
        ],
        out_shape=[
            jax.ShapeDtypeStruct((_N, _HID), jnp.float32),
            jax.ShapeDtypeStruct((_NS, _N, _F), jnp.float32),
        ],
    )(x, W1, b1.reshape(1, -1), W2, b2.reshape(1, -1), Wn)


def _mid_body(h_ref, s_ref, d_ref, Ws_ref, bg_ref, Wn_ref, h1_ref, g_ref):
    rdeg = 1.0 / jnp.maximum(d_ref[...][:, :1], 1.0)
    agg = s_ref[...] * rdeg
    h1 = jnp.dot(h_ref[...], Ws_ref[...], preferred_element_type=jnp.float32)
    h1 = jnp.maximum(h1 + agg + bg_ref[...], 0.0)
    h1_ref[...] = h1
    _split_g(jnp.dot(h1, Wn_ref[...], preferred_element_type=jnp.float32),
             g_ref)


def _mid(h, s, d, Ws, bg, Wn):
    return pl.pallas_call(
        _mid_body,
        grid=(_N // _R,),
        in_specs=[
            pl.BlockSpec((_R, _HID), lambda i: (i, 0)),
            pl.BlockSpec((_R, _HID), lambda i: (i, 0)),
            pl.BlockSpec((_R, _DW), lambda i: (i, 0)),
            pl.BlockSpec((_HID, _HID), lambda i: (0, 0)),
            pl.BlockSpec((1, _HID), lambda i: (0, 0)),
            pl.BlockSpec((_HID, _HID), lambda i: (0, 0)),
        ],
        out_specs=[
            pl.BlockSpec((_R, _HID), lambda i: (i, 0)),
            pl.BlockSpec((_NS, _R, _F), lambda i: (0, i, 0)),
        ],
        out_shape=[
            jax.ShapeDtypeStruct((_N, _HID), jnp.float32),
            jax.ShapeDtypeStruct((_NS, _N, _F), jnp.float32),
        ],
    )(h, s, d, Ws, bg.reshape(1, -1), Wn)


def _fin_body(h_ref, s_ref, d_ref, Ws_ref, bg_ref, Wc1_ref, bc1_ref, Wc2_ref,
              bc2_ref, out_ref, emb_ref):
    rdeg = 1.0 / jnp.maximum(d_ref[...][:, :1], 1.0)
    agg = s_ref[...] * rdeg
    emb = jnp.dot(h_ref[...], Ws_ref[...], preferred_element_type=jnp.float32)
    emb = jnp.maximum(emb + agg + bg_ref[...], 0.0)
    emb_ref[...] = emb
    t = jnp.dot(emb, Wc1_ref[...], preferred_element_type=jnp.float32)
    t = jnp.maximum(t + bc1_ref[...], 0.0)
    out_ref[...] = (
        jnp.dot(t, Wc2_ref[...], preferred_element_type=jnp.float32)
        + bc2_ref[...])


def _fin(h, s, d, Ws, bg, Wc1, bc1, Wc2, bc2):
    return pl.pallas_call(
        _fin_body,
        grid=(_N // _R,),
        in_specs=[
            pl.BlockSpec((_R, _HID), lambda i: (i, 0)),
            pl.BlockSpec((_R, _HID), lambda i: (i, 0)),
            pl.BlockSpec((_R, _DW), lambda i: (i, 0)),
            pl.BlockSpec((_HID, _HID), lambda i: (0, 0)),
            pl.BlockSpec((1, _HID), lambda i: (0, 0)),
            pl.BlockSpec((_HID, _HID), lambda i: (0, 0)),
            pl.BlockSpec((1, _HID), lambda i: (0, 0)),
            pl.BlockSpec((_HID, 128), lambda i: (0, 0)),
            pl.BlockSpec((1, 128), lambda i: (0, 0)),
        ],
        out_specs=[
            pl.BlockSpec((_R, 128), lambda i: (i, 0)),
            pl.BlockSpec((_R, _HID), lambda i: (i, 0)),
        ],
        out_shape=[
            jax.ShapeDtypeStruct((_N, 128), jnp.float32),
            jax.ShapeDtypeStruct((_N, _HID), jnp.float32),
        ],
    )(h, s, d, Ws, bg.reshape(1, -1), Wc1, bc1.reshape(1, -1), Wc2,
      bc2.reshape(1, -1))


def _glue_s(sagg):
    return sagg.transpose(0, 2, 1, 3).reshape(_NC * _NH, _HID)


def kernel(x, edge_index, W1, b1, W2, b2, Ws0, Wn0, bg0, Ws1, Wn1, bg1, Ws2,
           Wn2, bg2, Wc1, bc1, Wc2, bc2):
    _agg = _get_agg()
    _deg = _get_deg()
    src = edge_index[0]
    dst = edge_index[1]
    srcg = jnp.arange(_NS, dtype=jnp.int32)[:, None] * _N + src[None, :]
    dummy = _SKIP + (jnp.arange(_E, dtype=jnp.int32) % 8)
    d2 = []
    for c in range(_NC):
        local = dst - c * _NH
        inh = (local >= 0) & (local < _NH)
        mapped = local + jnp.where(local >= _SKIP, _SKW, 0)
        d2.append(jnp.where(inh, mapped, dummy))
    dst2 = jnp.stack(d2)

    d = _deg(dst2).reshape(_NC * _NH, _DW)
    h, g = _enc(x, W1, b1, W2, b2, Wn0)
    s = _glue_s(_agg(g.reshape(_NS * _N, _F), srcg, dst2))
    h, g = _mid(h, s, d, Ws0, bg0, Wn1)
    s = _glue_s(_agg(g.reshape(_NS * _N, _F), srcg, dst2))
    h, g = _mid(h, s, d, Ws1, bg1, Wn2)
    s = _glue_s(_agg(g.reshape(_NS * _N, _F), srcg, dst2))
    out, emb = _fin(h, s, d, Ws2, bg2, Wc1, bc1, Wc2, bc2)
    return (out, emb)
